# baseline (speedup 1.0000x reference)
import functools

import jax
import jax.numpy as jnp
from jax import lax
from jax.experimental import pallas as pl
from jax.experimental.pallas import tpu as pltpu


def _fused_kernel(x_ref, wqkv_ref, wo_ref, o_ref, wqkv_bf, wo_bf, *,
                  num_heads, dk, dv, L, eps, scale):
    nq = num_heads * dk

    @pl.when(pl.program_id(0) == 0)
    def _cast_weights():
        w = wqkv_ref[...]
        sc = jnp.where(
            lax.broadcasted_iota(jnp.int32, w.shape, 1) < nq, scale, 1.0)
        wqkv_bf[...] = (w * sc).astype(jnp.bfloat16)
        wo_bf[...] = wo_ref[...].astype(jnp.bfloat16)

    H2 = L // 2
    rowm = lax.broadcasted_iota(jnp.int32, (H2, H2), 0)
    colm = lax.broadcasted_iota(jnp.int32, (H2, H2), 1)
    tri = colm <= rowm
    causal_b = jnp.concatenate(
        [jnp.ones((H2, H2), jnp.bool_), tri], axis=-1)

    for i in range(x_ref.shape[0]):
        x = x_ref[i].astype(jnp.bfloat16)
        qkv = jnp.dot(x, wqkv_bf[...], preferred_element_type=jnp.float32)

        qkvb = qkv.astype(jnp.bfloat16)
        q = qkvb[:, :nq]
        k = qkvb[:, nq:2 * nq]
        v = qkvb[:, 2 * nq:]

        o_parts = []
        for h in range(num_heads):
            qh = q[:, h * dk:(h + 1) * dk]
            kh = k[:, h * dk:(h + 1) * dk]
            vh = v[:, h * dv:(h + 1) * dv]
            qa, qb = qh[:H2], qh[H2:]
            ka = kh[:H2]
            sa = lax.dot_general(qa, ka, (((1,), (1,)), ((), ())),
                                 preferred_element_type=jnp.float32)
            attna = 1.0 + sa + 0.5 * (sa * sa)
            attna = jnp.where(tri, attna, 0.0)
            za = jnp.sum(attna, axis=-1, keepdims=True)
            oa = jnp.dot(attna.astype(jnp.bfloat16), vh[:H2],
                         preferred_element_type=jnp.float32)
            sb = lax.dot_general(qb, kh, (((1,), (1,)), ((), ())),
                                 preferred_element_type=jnp.float32)
            attnb = 1.0 + sb + 0.5 * (sb * sb)
            attnb = jnp.where(causal_b, attnb, 0.0)
            zb = jnp.sum(attnb, axis=-1, keepdims=True)
            ob = jnp.dot(attnb.astype(jnp.bfloat16), vh,
                         preferred_element_type=jnp.float32)
            oh = jnp.concatenate(
                [oa * (1.0 / (za + eps)), ob * (1.0 / (zb + eps))], axis=0)
            o_parts.append(oh)
        o_norm = jnp.concatenate(o_parts, axis=-1).astype(jnp.bfloat16)

        o_ref[i] = jnp.dot(o_norm, wo_bf[...],
                           preferred_element_type=jnp.float32).astype(o_ref.dtype)


def kernel(Wqkv, Wo, x):
    B, L, D = x.shape
    num_heads = 8
    dk = 16
    nq = num_heads * dk
    nv = Wo.shape[0]
    dv = nv // num_heads
    eps = 1e-6
    scale = float(dk) ** -0.5

    body = functools.partial(_fused_kernel, num_heads=num_heads, dk=dk, dv=dv,
                             L=L, eps=eps, scale=scale)
    return pl.pallas_call(
        body,
        out_shape=jax.ShapeDtypeStruct((B, L, D), x.dtype),
        grid_spec=pltpu.PrefetchScalarGridSpec(
            num_scalar_prefetch=0,
            grid=(B // 2,),
            in_specs=[
                pl.BlockSpec((2, L, D), lambda b: (b, 0, 0)),
                pl.BlockSpec((D, 2 * nq + nv), lambda b: (0, 0)),
                pl.BlockSpec((nv, D), lambda b: (0, 0)),
            ],
            out_specs=pl.BlockSpec((2, L, D), lambda b: (b, 0, 0)),
            scratch_shapes=[
                pltpu.VMEM((D, 2 * nq + nv), jnp.bfloat16),
                pltpu.VMEM((nv, D), jnp.bfloat16),
            ],
        ),
        compiler_params=pltpu.CompilerParams(
            dimension_semantics=("arbitrary",)),
    )(x, Wqkv, Wo)


# speedup vs baseline: 1.0063x; 1.0063x over previous
import functools

import jax
import jax.numpy as jnp
from jax import lax
from jax.experimental import pallas as pl
from jax.experimental.pallas import tpu as pltpu


def _fused_kernel(x_ref, wqkv_ref, wo_ref, o_ref, wqkv_bf, wo_bf, *,
                  num_heads, dk, dv, L, eps, scale):
    nq = num_heads * dk

    @pl.when(pl.program_id(0) == 0)
    def _cast_weights():
        w = wqkv_ref[...]
        sc = jnp.where(
            lax.broadcasted_iota(jnp.int32, w.shape, 1) < nq, scale, 1.0)
        wqkv_bf[...] = (w * sc).astype(jnp.bfloat16)
        wo_bf[...] = wo_ref[...].astype(jnp.bfloat16)

    H2 = L // 2
    rowm = lax.broadcasted_iota(jnp.int32, (H2, H2), 0)
    colm = lax.broadcasted_iota(jnp.int32, (H2, H2), 1)
    tri = colm <= rowm
    causal_b = jnp.concatenate(
        [jnp.ones((H2, H2), jnp.bool_), tri], axis=-1)

    nb = x_ref.shape[0]
    x2 = x_ref[...].reshape(nb * L, x_ref.shape[2]).astype(jnp.bfloat16)
    qkv = jnp.dot(x2, wqkv_bf[...], preferred_element_type=jnp.float32)
    qkvb = qkv.astype(jnp.bfloat16)

    o_norm_parts = []
    for i in range(nb):
        q = qkvb[i * L:(i + 1) * L, :nq]
        k = qkvb[i * L:(i + 1) * L, nq:2 * nq]
        v = qkvb[i * L:(i + 1) * L, 2 * nq:]

        o_parts = []
        for h in range(num_heads):
            qh = q[:, h * dk:(h + 1) * dk]
            kh = k[:, h * dk:(h + 1) * dk]
            vh = v[:, h * dv:(h + 1) * dv]
            qa, qb = qh[:H2], qh[H2:]
            ka = kh[:H2]
            sa = lax.dot_general(qa, ka, (((1,), (1,)), ((), ())),
                                 preferred_element_type=jnp.float32)
            attna = 1.0 + sa + 0.5 * (sa * sa)
            attna = jnp.where(tri, attna, 0.0)
            za = jnp.sum(attna, axis=-1, keepdims=True)
            oa = jnp.dot(attna.astype(jnp.bfloat16), vh[:H2],
                         preferred_element_type=jnp.float32)
            sb = lax.dot_general(qb, kh, (((1,), (1,)), ((), ())),
                                 preferred_element_type=jnp.float32)
            attnb = 1.0 + sb + 0.5 * (sb * sb)
            attnb = jnp.where(causal_b, attnb, 0.0)
            zb = jnp.sum(attnb, axis=-1, keepdims=True)
            ob = jnp.dot(attnb.astype(jnp.bfloat16), vh,
                         preferred_element_type=jnp.float32)
            oh = jnp.concatenate(
                [oa * (1.0 / (za + eps)), ob * (1.0 / (zb + eps))], axis=0)
            o_parts.append(oh)
        o_norm_parts.append(
            jnp.concatenate(o_parts, axis=-1).astype(jnp.bfloat16))
    o_norm2 = jnp.concatenate(o_norm_parts, axis=0)

    out = jnp.dot(o_norm2, wo_bf[...],
                  preferred_element_type=jnp.float32)
    o_ref[...] = out.reshape(o_ref.shape).astype(o_ref.dtype)


def kernel(Wqkv, Wo, x):
    B, L, D = x.shape
    num_heads = 8
    dk = 16
    nq = num_heads * dk
    nv = Wo.shape[0]
    dv = nv // num_heads
    eps = 1e-6
    scale = float(dk) ** -0.5

    body = functools.partial(_fused_kernel, num_heads=num_heads, dk=dk, dv=dv,
                             L=L, eps=eps, scale=scale)
    return pl.pallas_call(
        body,
        out_shape=jax.ShapeDtypeStruct((B, L, D), x.dtype),
        grid_spec=pltpu.PrefetchScalarGridSpec(
            num_scalar_prefetch=0,
            grid=(B // 2,),
            in_specs=[
                pl.BlockSpec((2, L, D), lambda b: (b, 0, 0)),
                pl.BlockSpec((D, 2 * nq + nv), lambda b: (0, 0)),
                pl.BlockSpec((nv, D), lambda b: (0, 0)),
            ],
            out_specs=pl.BlockSpec((2, L, D), lambda b: (b, 0, 0)),
            scratch_shapes=[
                pltpu.VMEM((D, 2 * nq + nv), jnp.bfloat16),
                pltpu.VMEM((nv, D), jnp.bfloat16),
            ],
        ),
        compiler_params=pltpu.CompilerParams(
            dimension_semantics=("arbitrary",)),
    )(x, Wqkv, Wo)


# speedup vs baseline: 1.0190x; 1.0126x over previous
import functools

import jax
import jax.numpy as jnp
from jax import lax
from jax.experimental import pallas as pl
from jax.experimental.pallas import tpu as pltpu


def _fused_kernel(x_ref, wqkv_ref, wo_ref, o_ref, wqkv_bf, wo_bf, *,
                  num_heads, dk, dv, L, eps, scale):
    nq = num_heads * dk

    @pl.when(pl.program_id(0) == 0)
    def _cast_weights():
        w = wqkv_ref[...]
        sc = jnp.where(
            lax.broadcasted_iota(jnp.int32, w.shape, 1) < nq, scale, 1.0)
        wqkv_bf[...] = (w * sc).astype(jnp.bfloat16)
        wo_bf[...] = wo_ref[...].astype(jnp.bfloat16)

    H2 = L // 2
    rowm = lax.broadcasted_iota(jnp.int32, (H2, H2), 0)
    colm = lax.broadcasted_iota(jnp.int32, (H2, H2), 1)
    tri = colm <= rowm
    causal_b = jnp.concatenate(
        [jnp.ones((H2, H2), jnp.bool_), tri], axis=-1)

    nb = x_ref.shape[0]
    x2 = x_ref[...].reshape(nb * L, x_ref.shape[2]).astype(jnp.bfloat16)
    qkv = jnp.dot(x2, wqkv_bf[...], preferred_element_type=jnp.float32)
    qkvb = qkv.astype(jnp.bfloat16)

    o_norm_parts = []
    for i in range(nb):
        q = qkvb[i * L:(i + 1) * L, :nq]
        k = qkvb[i * L:(i + 1) * L, nq:2 * nq]
        v = qkvb[i * L:(i + 1) * L, 2 * nq:]

        o_parts = []
        for h in range(num_heads):
            qh = q[:, h * dk:(h + 1) * dk]
            kh = k[:, h * dk:(h + 1) * dk]
            vh = v[:, h * dv:(h + 1) * dv]
            qa, qb = qh[:H2], qh[H2:]
            ka = kh[:H2]
            sa = lax.dot_general(qa, ka, (((1,), (1,)), ((), ())),
                                 preferred_element_type=jnp.float32)
            ua = sa + 1.0
            attna = ua * ua + 1.0
            attna = jnp.where(tri, attna, 0.0)
            za = jnp.sum(attna, axis=-1, keepdims=True)
            oa = jnp.dot(attna.astype(jnp.bfloat16), vh[:H2],
                         preferred_element_type=jnp.float32)
            sb = lax.dot_general(qb, kh, (((1,), (1,)), ((), ())),
                                 preferred_element_type=jnp.float32)
            ub = sb + 1.0
            attnb = ub * ub + 1.0
            attnb = jnp.where(causal_b, attnb, 0.0)
            zb = jnp.sum(attnb, axis=-1, keepdims=True)
            ob = jnp.dot(attnb.astype(jnp.bfloat16), vh,
                         preferred_element_type=jnp.float32)
            oh = jnp.concatenate(
                [oa * (1.0 / (za + eps)), ob * (1.0 / (zb + eps))], axis=0)
            o_parts.append(oh)
        o_norm_parts.append(
            jnp.concatenate(o_parts, axis=-1).astype(jnp.bfloat16))
    o_norm2 = jnp.concatenate(o_norm_parts, axis=0)

    out = jnp.dot(o_norm2, wo_bf[...],
                  preferred_element_type=jnp.float32)
    o_ref[...] = out.reshape(o_ref.shape).astype(o_ref.dtype)


def kernel(Wqkv, Wo, x):
    B, L, D = x.shape
    num_heads = 8
    dk = 16
    nq = num_heads * dk
    nv = Wo.shape[0]
    dv = nv // num_heads
    eps = 1e-6
    scale = float(dk) ** -0.5

    body = functools.partial(_fused_kernel, num_heads=num_heads, dk=dk, dv=dv,
                             L=L, eps=eps, scale=scale)
    return pl.pallas_call(
        body,
        out_shape=jax.ShapeDtypeStruct((B, L, D), x.dtype),
        grid_spec=pltpu.PrefetchScalarGridSpec(
            num_scalar_prefetch=0,
            grid=(B // 2,),
            in_specs=[
                pl.BlockSpec((2, L, D), lambda b: (b, 0, 0)),
                pl.BlockSpec((D, 2 * nq + nv), lambda b: (0, 0)),
                pl.BlockSpec((nv, D), lambda b: (0, 0)),
            ],
            out_specs=pl.BlockSpec((2, L, D), lambda b: (b, 0, 0)),
            scratch_shapes=[
                pltpu.VMEM((D, 2 * nq + nv), jnp.bfloat16),
                pltpu.VMEM((nv, D), jnp.bfloat16),
            ],
        ),
        compiler_params=pltpu.CompilerParams(
            dimension_semantics=("arbitrary",)),
    )(x, Wqkv, Wo)
